# baseline (speedup 1.0000x reference)
import functools

import jax
import jax.numpy as jnp
from jax import lax
from jax.experimental import pallas as pl
from jax.experimental.pallas import tpu as pltpu
from jax.experimental.pallas import tpu_sc as plsc

NC = 2
NS = 16
NW = NC * NS
RPB = 128
NBUF = 5


def _gather_fn(nstep, d):
    mesh = plsc.VectorSubcoreMesh(
        core_axis_name="c", subcore_axis_name="s",
        num_cores=NC, num_subcores=NS,
    )

    @functools.partial(
        pl.kernel,
        out_type=jax.ShapeDtypeStruct((NW * nstep * RPB, d), jnp.float32),
        mesh=mesh,
        compiler_params=pltpu.CompilerParams(use_tc_tiling_on_sc=False),
        scratch_types=[
            pltpu.VMEM((nstep, RPB), jnp.int32),
            pltpu.VMEM((NBUF, RPB, d), jnp.float32),
            pltpu.SemaphoreType.DMA,
            pltpu.SemaphoreType.DMA,
        ],
    )
    def gather_kernel(ids_hbm, table_hbm, out_hbm, idx_v, bufs, gsem, ssem):
        wid = lax.axis_index("s") * NC + lax.axis_index("c")
        base = wid * nstep * RPB
        pltpu.sync_copy(ids_hbm.at[wid], idx_v)

        for b in range(NBUF):
            pltpu.async_copy(table_hbm.at[idx_v.at[b]], bufs.at[b], gsem)

        @pl.loop(0, nstep, step=NBUF)
        def _(g):
            for b in range(NBUF):
                j = g + b
                pltpu.make_async_copy(
                    table_hbm.at[idx_v.at[0]], bufs.at[b], gsem
                ).wait()
                st = pltpu.async_copy(
                    bufs.at[b], out_hbm.at[pl.ds(base + j * RPB, RPB)], ssem
                )
                st.wait()
                @pl.when(j + NBUF < nstep)
                def _():
                    pltpu.async_copy(
                        table_hbm.at[idx_v.at[j + NBUF]], bufs.at[b], gsem
                    )

    return gather_kernel


def kernel(input_ids, table):
    batch, seq = input_ids.shape
    _, d = table.shape
    n = batch * seq
    assert n % (NW * RPB) == 0
    nstep = n // (NW * RPB)
    assert nstep % NBUF == 0
    ids = input_ids.reshape(NW, nstep, RPB)
    out = _gather_fn(nstep, d)(ids, table)
    return out.reshape(batch, seq, d)
